# baseline (speedup 1.0000x reference)
import functools

import jax
import jax.numpy as jnp
from jax.experimental import pallas as pl
from jax.experimental.pallas import tpu as pltpu

_E = 8
_K = 2
_D = 768
_TILE = 512


def _moe_body(x_ref, gate_w_ref, gate_b_ref,
              fc1_w_ref, fc1_b_ref, ln1_g_ref, ln1_b_ref,
              fc2_w_ref, fc2_b_ref, ln2_g_ref, ln2_b_ref,
              out_ref, top_ref, bot_ref, sq_ref,
              acc_top, acc_bot, wt_ref, wb_ref):
    e = pl.program_id(1)
    x = x_ref[...]

    @pl.when(e == 0)
    def _init():
        eye8 = (jax.lax.broadcasted_iota(jnp.int32, (_E, _E), 0) ==
                jax.lax.broadcasted_iota(jnp.int32, (_E, _E), 1)
                ).astype(jnp.float32)
        scores = jax.lax.dot_general(
            x, gate_w_ref[...], (((1,), (1,)), ((), ())),
            preferred_element_type=jnp.float32)
        st = jax.lax.dot_general(
            eye8, scores, (((1,), (1,)), ((), ())),
            precision=jax.lax.Precision.HIGHEST,
            preferred_element_type=jnp.float32) + gate_b_ref[...]
        it = jax.lax.broadcasted_iota(jnp.int32, (_E, _TILE), 0)
        rank_t = jnp.zeros((_E, _TILE), jnp.float32)
        rank_b = jnp.zeros((_E, _TILE), jnp.float32)
        for j in range(_E):
            sj = st[j:j + 1, :]
            eq_earlier = (sj == st) & (it > j)
            rank_t += ((sj > st) | eq_earlier).astype(jnp.float32)
            rank_b += ((sj < st) | eq_earlier).astype(jnp.float32)
        is_top = rank_t < _K
        is_bot = rank_b < _K

        neg_inf = jnp.float32(-1e30)
        m_top = jnp.max(jnp.where(is_top, st, neg_inf), axis=0, keepdims=True)
        e_top = jnp.where(is_top, jnp.exp(st - m_top), 0.0)
        wtT = e_top / jnp.sum(e_top, axis=0, keepdims=True)

        m_bot = jnp.max(jnp.where(is_bot, st, neg_inf), axis=0, keepdims=True)
        e_bot = jnp.where(is_bot, jnp.exp(st - m_bot), 0.0)
        wbT = e_bot / jnp.sum(e_bot, axis=0, keepdims=True)

        wt_ref[...] = jax.lax.dot_general(
            wtT, eye8, (((0,), (0,)), ((), ())),
            precision=jax.lax.Precision.HIGHEST,
            preferred_element_type=jnp.float32)
        wb_ref[...] = jax.lax.dot_general(
            wbT, eye8, (((0,), (0,)), ((), ())),
            precision=jax.lax.Precision.HIGHEST,
            preferred_element_type=jnp.float32)
        acc_top[...] = jnp.zeros_like(acc_top)
        acc_bot[...] = jnp.zeros_like(acc_bot)

    w1 = fc1_w_ref[e]
    h = jax.lax.dot_general(x, w1, (((1,), (1,)), ((), ())),
                            preferred_element_type=jnp.float32)
    h = h + fc1_b_ref[e]
    mu = jnp.mean(h, axis=1, keepdims=True)
    var = jnp.mean((h - mu) * (h - mu), axis=1, keepdims=True)
    h = (h - mu) * jax.lax.rsqrt(var + 1e-5) * ln1_g_ref[e] + ln1_b_ref[e]
    h = jnp.maximum(h, 0.0)

    w2 = fc2_w_ref[e]
    o = jax.lax.dot_general(h, w2, (((1,), (1,)), ((), ())),
                            preferred_element_type=jnp.float32)
    o = o + fc2_b_ref[e]
    mu2 = jnp.mean(o, axis=1, keepdims=True)
    var2 = jnp.mean((o - mu2) * (o - mu2), axis=1, keepdims=True)
    y = (o - mu2) * jax.lax.rsqrt(var2 + 1e-5) * ln2_g_ref[e] + ln2_b_ref[e]

    onehot = (jax.lax.broadcasted_iota(jnp.int32, (1, _E), 1) == e
              ).astype(jnp.float32)
    wt_col = jnp.sum(wt_ref[...] * onehot, axis=1, keepdims=True)
    wb_col = jnp.sum(wb_ref[...] * onehot, axis=1, keepdims=True)
    acc_top[...] += wt_col * y
    acc_bot[...] += wb_col * y

    @pl.when(e == _E - 1)
    def _finish():
        top = acc_top[...]
        bot = acc_bot[...]
        top_ref[...] = top
        bot_ref[...] = bot
        out_ref[...] = top + x
        d = top - bot
        sq = jnp.sum(d * d)
        sq_ref[...] = jnp.full((1, 1, 128), sq, dtype=jnp.float32)


@functools.partial(jax.jit, static_argnames=())
def kernel(x, gate_w, gate_b, fc1_w, fc1_b, ln1_g, ln1_b,
           fc2_w, fc2_b, ln2_g, ln2_b):
    b, n, d = x.shape
    bn = b * n
    xf = x.reshape(bn, d)
    num_tiles = bn // _TILE
    gate_b2 = gate_b.reshape(_E, 1)
    fc1_b3 = fc1_b.reshape(_E, 1, d)
    ln1_g3 = ln1_g.reshape(_E, 1, d)
    ln1_b3 = ln1_b.reshape(_E, 1, d)
    fc2_b3 = fc2_b.reshape(_E, 1, d)
    ln2_g3 = ln2_g.reshape(_E, 1, d)
    ln2_b3 = ln2_b.reshape(_E, 1, d)

    grid = (num_tiles, _E)

    def t_only(t, e):
        return (t, 0)

    def const3(t, e):
        return (0, 0, 0)

    def const2(t, e):
        return (0, 0)

    out, top, bot, sq = pl.pallas_call(
        _moe_body,
        grid=grid,
        in_specs=[
            pl.BlockSpec((_TILE, d), t_only),
            pl.BlockSpec((_E, d), const2),
            pl.BlockSpec((_E, 1), const2),
            pl.BlockSpec((_E, d, d), const3),
            pl.BlockSpec((_E, 1, d), const3),
            pl.BlockSpec((_E, 1, d), const3),
            pl.BlockSpec((_E, 1, d), const3),
            pl.BlockSpec((_E, d, d), const3),
            pl.BlockSpec((_E, 1, d), const3),
            pl.BlockSpec((_E, 1, d), const3),
            pl.BlockSpec((_E, 1, d), const3),
        ],
        out_specs=[
            pl.BlockSpec((_TILE, d), t_only),
            pl.BlockSpec((_TILE, d), t_only),
            pl.BlockSpec((_TILE, d), t_only),
            pl.BlockSpec((1, 1, 128), lambda t, e: (t, 0, 0)),
        ],
        out_shape=[
            jax.ShapeDtypeStruct((bn, d), jnp.float32),
            jax.ShapeDtypeStruct((bn, d), jnp.float32),
            jax.ShapeDtypeStruct((bn, d), jnp.float32),
            jax.ShapeDtypeStruct((num_tiles, 1, 128), jnp.float32),
        ],
        scratch_shapes=[
            pltpu.VMEM((_TILE, d), jnp.float32),
            pltpu.VMEM((_TILE, d), jnp.float32),
            pltpu.VMEM((_TILE, _E), jnp.float32),
            pltpu.VMEM((_TILE, _E), jnp.float32),
        ],
        compiler_params=pltpu.CompilerParams(
            dimension_semantics=("parallel", "arbitrary"),
        ),
    )(xf, gate_w, gate_b2, fc1_w, fc1_b3, ln1_g3, ln1_b3,
      fc2_w, fc2_b3, ln2_g3, ln2_b3)

    output = out.reshape(b, n, d)
    output_top = top.reshape(b, n, d)
    output_bottom = bot.reshape(b, n, d)

    tiles_per_b = num_tiles // b
    sq_b = sq[:, 0, 0].reshape(b, tiles_per_b).sum(axis=1)
    dist = jnp.sqrt(sq_b)
    orthogonality_loss = (1.0 / (dist + 1e-8)).mean()
    return (output, output_top, output_bottom, orthogonality_loss)


# speedup vs baseline: 1.0688x; 1.0688x over previous
import functools

import jax
import jax.numpy as jnp
from jax.experimental import pallas as pl
from jax.experimental.pallas import tpu as pltpu

_E = 8
_K = 2
_D = 768
_TILE = 512


def _moe_body(x_ref, gate_w_ref, gate_b_ref,
              fc1_w_ref, fc1_b_ref, ln1_g_ref, ln1_b_ref,
              fc2_w_ref, fc2_b_ref, ln2_g_ref, ln2_b_ref,
              out_ref, top_ref, bot_ref, sq_ref,
              acc_top, acc_bot, wt_ref, wb_ref, o_ref, g_ref, b_ref):
    e = pl.program_id(1)
    x = x_ref[...]

    @pl.when(e == 0)
    def _init():
        eye8 = (jax.lax.broadcasted_iota(jnp.int32, (_E, _E), 0) ==
                jax.lax.broadcasted_iota(jnp.int32, (_E, _E), 1)
                ).astype(jnp.float32)
        scores = jax.lax.dot_general(
            x, gate_w_ref[...], (((1,), (1,)), ((), ())),
            preferred_element_type=jnp.float32)
        st = jax.lax.dot_general(
            eye8, scores, (((1,), (1,)), ((), ())),
            precision=jax.lax.Precision.HIGHEST,
            preferred_element_type=jnp.float32) + gate_b_ref[...]
        it = jax.lax.broadcasted_iota(jnp.int32, (_E, _TILE), 0)
        rank_t = jnp.zeros((_E, _TILE), jnp.float32)
        rank_b = jnp.zeros((_E, _TILE), jnp.float32)
        for j in range(_E):
            sj = st[j:j + 1, :]
            eq_earlier = (sj == st) & (it > j)
            rank_t += ((sj > st) | eq_earlier).astype(jnp.float32)
            rank_b += ((sj < st) | eq_earlier).astype(jnp.float32)
        is_top = rank_t < _K
        is_bot = rank_b < _K

        neg_inf = jnp.float32(-1e30)
        m_top = jnp.max(jnp.where(is_top, st, neg_inf), axis=0, keepdims=True)
        e_top = jnp.where(is_top, jnp.exp(st - m_top), 0.0)
        wtT = e_top / jnp.sum(e_top, axis=0, keepdims=True)

        m_bot = jnp.max(jnp.where(is_bot, st, neg_inf), axis=0, keepdims=True)
        e_bot = jnp.where(is_bot, jnp.exp(st - m_bot), 0.0)
        wbT = e_bot / jnp.sum(e_bot, axis=0, keepdims=True)

        wt_ref[...] = jax.lax.dot_general(
            wtT, eye8, (((0,), (0,)), ((), ())),
            precision=jax.lax.Precision.HIGHEST,
            preferred_element_type=jnp.float32)
        wb_ref[...] = jax.lax.dot_general(
            wbT, eye8, (((0,), (0,)), ((), ())),
            precision=jax.lax.Precision.HIGHEST,
            preferred_element_type=jnp.float32)

    def combine(oe, g2, b2, eid, first):
        mu2 = jnp.mean(oe, axis=1, keepdims=True)
        var2 = jnp.mean((oe - mu2) * (oe - mu2), axis=1, keepdims=True)
        y = (oe - mu2) * jax.lax.rsqrt(var2 + 1e-5) * g2 + b2
        onehot = (jax.lax.broadcasted_iota(jnp.int32, (1, _E), 1) == eid
                  ).astype(jnp.float32)
        wt_col = jnp.sum(wt_ref[...] * onehot, axis=1, keepdims=True)
        wb_col = jnp.sum(wb_ref[...] * onehot, axis=1, keepdims=True)
        valid = eid >= 0
        ct = jnp.where(valid, wt_col * y, 0.0)
        cb = jnp.where(valid, wb_col * y, 0.0)
        zero = jnp.float32(0.0)
        acc_top[...] = jnp.where(first, zero, acc_top[...]) + ct
        acc_bot[...] = jnp.where(first, zero, acc_bot[...]) + cb

    combine(o_ref[...], g_ref[...], b_ref[...], e - 1, e == 0)

    w1 = fc1_w_ref[0]
    h = jax.lax.dot_general(x, w1, (((1,), (1,)), ((), ())),
                            preferred_element_type=jnp.float32)
    h = h + fc1_b_ref[0]
    mu = jnp.mean(h, axis=1, keepdims=True)
    var = jnp.mean((h - mu) * (h - mu), axis=1, keepdims=True)
    h = (h - mu) * jax.lax.rsqrt(var + 1e-5) * ln1_g_ref[0] + ln1_b_ref[0]
    h = jnp.maximum(h, 0.0)

    w2 = fc2_w_ref[0]
    o = jax.lax.dot_general(h, w2, (((1,), (1,)), ((), ())),
                            preferred_element_type=jnp.float32)
    o_ref[...] = o + fc2_b_ref[0]
    g_ref[...] = ln2_g_ref[0]
    b_ref[...] = ln2_b_ref[0]

    @pl.when(e == _E - 1)
    def _finish():
        combine(o_ref[...], ln2_g_ref[0], ln2_b_ref[0], _E - 1, False)
        top = acc_top[...]
        bot = acc_bot[...]
        top_ref[...] = top
        bot_ref[...] = bot
        out_ref[...] = top + x
        d = top - bot
        sq = jnp.sum(d * d)
        sq_ref[...] = jnp.full((1, 1, 128), sq, dtype=jnp.float32)


@functools.partial(jax.jit, static_argnames=())
def kernel(x, gate_w, gate_b, fc1_w, fc1_b, ln1_g, ln1_b,
           fc2_w, fc2_b, ln2_g, ln2_b):
    b, n, d = x.shape
    bn = b * n
    xf = x.reshape(bn, d)
    num_tiles = bn // _TILE
    gate_b2 = gate_b.reshape(_E, 1)
    fc1_b3 = fc1_b.reshape(_E, 1, d)
    ln1_g3 = ln1_g.reshape(_E, 1, d)
    ln1_b3 = ln1_b.reshape(_E, 1, d)
    fc2_b3 = fc2_b.reshape(_E, 1, d)
    ln2_g3 = ln2_g.reshape(_E, 1, d)
    ln2_b3 = ln2_b.reshape(_E, 1, d)

    grid = (num_tiles, _E)

    def t_only(t, e):
        return (t, 0)

    def e_row3(t, e):
        return (e, 0, 0)

    def const2(t, e):
        return (0, 0)

    out, top, bot, sq = pl.pallas_call(
        _moe_body,
        grid=grid,
        in_specs=[
            pl.BlockSpec((_TILE, d), t_only),
            pl.BlockSpec((_E, d), const2),
            pl.BlockSpec((_E, 1), const2),
            pl.BlockSpec((1, d, d), e_row3),
            pl.BlockSpec((1, 1, d), e_row3),
            pl.BlockSpec((1, 1, d), e_row3),
            pl.BlockSpec((1, 1, d), e_row3),
            pl.BlockSpec((1, d, d), e_row3),
            pl.BlockSpec((1, 1, d), e_row3),
            pl.BlockSpec((1, 1, d), e_row3),
            pl.BlockSpec((1, 1, d), e_row3),
        ],
        out_specs=[
            pl.BlockSpec((_TILE, d), t_only),
            pl.BlockSpec((_TILE, d), t_only),
            pl.BlockSpec((_TILE, d), t_only),
            pl.BlockSpec((1, 1, 128), lambda t, e: (t, 0, 0)),
        ],
        out_shape=[
            jax.ShapeDtypeStruct((bn, d), jnp.float32),
            jax.ShapeDtypeStruct((bn, d), jnp.float32),
            jax.ShapeDtypeStruct((bn, d), jnp.float32),
            jax.ShapeDtypeStruct((num_tiles, 1, 128), jnp.float32),
        ],
        scratch_shapes=[
            pltpu.VMEM((_TILE, _D), jnp.float32),
            pltpu.VMEM((_TILE, _D), jnp.float32),
            pltpu.VMEM((_TILE, _E), jnp.float32),
            pltpu.VMEM((_TILE, _E), jnp.float32),
            pltpu.VMEM((_TILE, _D), jnp.float32),
            pltpu.VMEM((1, _D), jnp.float32),
            pltpu.VMEM((1, _D), jnp.float32),
        ],
        compiler_params=pltpu.CompilerParams(
            dimension_semantics=("parallel", "arbitrary"),
        ),
    )(xf, gate_w, gate_b2, fc1_w, fc1_b3, ln1_g3, ln1_b3,
      fc2_w, fc2_b3, ln2_g3, ln2_b3)

    output = out.reshape(b, n, d)
    output_top = top.reshape(b, n, d)
    output_bottom = bot.reshape(b, n, d)

    tiles_per_b = num_tiles // b
    sq_b = sq[:, 0, 0].reshape(b, tiles_per_b).sum(axis=1)
    dist = jnp.sqrt(sq_b)
    orthogonality_loss = (1.0 / (dist + 1e-8)).mean()
    return (output, output_top, output_bottom, orthogonality_loss)


# speedup vs baseline: 1.1508x; 1.0766x over previous
import functools

import jax
import jax.numpy as jnp
from jax.experimental import pallas as pl
from jax.experimental.pallas import tpu as pltpu

_E = 8
_K = 2
_D = 768
_TILE = 1024


def _moe_body(x_ref, gate_w_ref, gate_b_ref,
              fc1_w_ref, fc1_b_ref, ln1_g_ref, ln1_b_ref,
              fc2_w_ref, fc2_b_ref, ln2_g_ref, ln2_b_ref,
              out_ref, top_ref, bot_ref, sq_ref,
              acc_top, acc_bot, wt_ref, wb_ref, o_ref, g_ref, b_ref):
    e = pl.program_id(1)
    x = x_ref[...]

    @pl.when(e == 0)
    def _init():
        eye8 = (jax.lax.broadcasted_iota(jnp.int32, (_E, _E), 0) ==
                jax.lax.broadcasted_iota(jnp.int32, (_E, _E), 1)
                ).astype(jnp.float32)
        scores = jax.lax.dot_general(
            x, gate_w_ref[...], (((1,), (1,)), ((), ())),
            preferred_element_type=jnp.float32)
        st = jax.lax.dot_general(
            eye8, scores, (((1,), (1,)), ((), ())),
            precision=jax.lax.Precision.HIGHEST,
            preferred_element_type=jnp.float32) + gate_b_ref[...]
        it = jax.lax.broadcasted_iota(jnp.int32, (_E, _TILE), 0)
        rank_t = jnp.zeros((_E, _TILE), jnp.float32)
        rank_b = jnp.zeros((_E, _TILE), jnp.float32)
        for j in range(_E):
            sj = st[j:j + 1, :]
            eq_earlier = (sj == st) & (it > j)
            rank_t += ((sj > st) | eq_earlier).astype(jnp.float32)
            rank_b += ((sj < st) | eq_earlier).astype(jnp.float32)
        is_top = rank_t < _K
        is_bot = rank_b < _K

        neg_inf = jnp.float32(-1e30)
        m_top = jnp.max(jnp.where(is_top, st, neg_inf), axis=0, keepdims=True)
        e_top = jnp.where(is_top, jnp.exp(st - m_top), 0.0)
        wtT = e_top / jnp.sum(e_top, axis=0, keepdims=True)

        m_bot = jnp.max(jnp.where(is_bot, st, neg_inf), axis=0, keepdims=True)
        e_bot = jnp.where(is_bot, jnp.exp(st - m_bot), 0.0)
        wbT = e_bot / jnp.sum(e_bot, axis=0, keepdims=True)

        wt_ref[...] = jax.lax.dot_general(
            wtT, eye8, (((0,), (0,)), ((), ())),
            precision=jax.lax.Precision.HIGHEST,
            preferred_element_type=jnp.float32)
        wb_ref[...] = jax.lax.dot_general(
            wbT, eye8, (((0,), (0,)), ((), ())),
            precision=jax.lax.Precision.HIGHEST,
            preferred_element_type=jnp.float32)

    def combine(oe, g2, b2, eid, first):
        mu2 = jnp.mean(oe, axis=1, keepdims=True)
        var2 = jnp.mean((oe - mu2) * (oe - mu2), axis=1, keepdims=True)
        y = (oe - mu2) * jax.lax.rsqrt(var2 + 1e-5) * g2 + b2
        onehot = (jax.lax.broadcasted_iota(jnp.int32, (1, _E), 1) == eid
                  ).astype(jnp.float32)
        wt_col = jnp.sum(wt_ref[...] * onehot, axis=1, keepdims=True)
        wb_col = jnp.sum(wb_ref[...] * onehot, axis=1, keepdims=True)
        valid = eid >= 0
        ct = jnp.where(valid, wt_col * y, 0.0)
        cb = jnp.where(valid, wb_col * y, 0.0)
        zero = jnp.float32(0.0)
        acc_top[...] = jnp.where(first, zero, acc_top[...]) + ct
        acc_bot[...] = jnp.where(first, zero, acc_bot[...]) + cb

    combine(o_ref[...], g_ref[...], b_ref[...], e - 1, e == 0)

    w1 = fc1_w_ref[0]
    h = jax.lax.dot_general(x, w1, (((1,), (1,)), ((), ())),
                            preferred_element_type=jnp.float32)
    h = h + fc1_b_ref[0]
    mu = jnp.mean(h, axis=1, keepdims=True)
    var = jnp.mean((h - mu) * (h - mu), axis=1, keepdims=True)
    h = (h - mu) * jax.lax.rsqrt(var + 1e-5) * ln1_g_ref[0] + ln1_b_ref[0]
    h = jnp.maximum(h, 0.0)

    w2 = fc2_w_ref[0]
    o = jax.lax.dot_general(h, w2, (((1,), (1,)), ((), ())),
                            preferred_element_type=jnp.float32)
    o_ref[...] = o + fc2_b_ref[0]
    g_ref[...] = ln2_g_ref[0]
    b_ref[...] = ln2_b_ref[0]

    @pl.when(e == _E - 1)
    def _finish():
        combine(o_ref[...], ln2_g_ref[0], ln2_b_ref[0], _E - 1, False)
        top = acc_top[...]
        bot = acc_bot[...]
        top_ref[...] = top
        bot_ref[...] = bot
        out_ref[...] = top + x
        d = top - bot
        sq = jnp.sum(d * d)
        sq_ref[...] = jnp.full((1, 1, 128), sq, dtype=jnp.float32)


@functools.partial(jax.jit, static_argnames=())
def kernel(x, gate_w, gate_b, fc1_w, fc1_b, ln1_g, ln1_b,
           fc2_w, fc2_b, ln2_g, ln2_b):
    b, n, d = x.shape
    bn = b * n
    xf = x.reshape(bn, d)
    num_tiles = bn // _TILE
    gate_b2 = gate_b.reshape(_E, 1)
    fc1_b3 = fc1_b.reshape(_E, 1, d)
    ln1_g3 = ln1_g.reshape(_E, 1, d)
    ln1_b3 = ln1_b.reshape(_E, 1, d)
    fc2_b3 = fc2_b.reshape(_E, 1, d)
    ln2_g3 = ln2_g.reshape(_E, 1, d)
    ln2_b3 = ln2_b.reshape(_E, 1, d)

    grid = (num_tiles, _E)

    def t_only(t, e):
        return (t, 0)

    def e_row3(t, e):
        return (e, 0, 0)

    def const2(t, e):
        return (0, 0)

    out, top, bot, sq = pl.pallas_call(
        _moe_body,
        grid=grid,
        in_specs=[
            pl.BlockSpec((_TILE, d), t_only),
            pl.BlockSpec((_E, d), const2),
            pl.BlockSpec((_E, 1), const2),
            pl.BlockSpec((1, d, d), e_row3),
            pl.BlockSpec((1, 1, d), e_row3),
            pl.BlockSpec((1, 1, d), e_row3),
            pl.BlockSpec((1, 1, d), e_row3),
            pl.BlockSpec((1, d, d), e_row3),
            pl.BlockSpec((1, 1, d), e_row3),
            pl.BlockSpec((1, 1, d), e_row3),
            pl.BlockSpec((1, 1, d), e_row3),
        ],
        out_specs=[
            pl.BlockSpec((_TILE, d), t_only),
            pl.BlockSpec((_TILE, d), t_only),
            pl.BlockSpec((_TILE, d), t_only),
            pl.BlockSpec((1, 1, 128), lambda t, e: (t, 0, 0)),
        ],
        out_shape=[
            jax.ShapeDtypeStruct((bn, d), jnp.float32),
            jax.ShapeDtypeStruct((bn, d), jnp.float32),
            jax.ShapeDtypeStruct((bn, d), jnp.float32),
            jax.ShapeDtypeStruct((num_tiles, 1, 128), jnp.float32),
        ],
        scratch_shapes=[
            pltpu.VMEM((_TILE, _D), jnp.float32),
            pltpu.VMEM((_TILE, _D), jnp.float32),
            pltpu.VMEM((_TILE, _E), jnp.float32),
            pltpu.VMEM((_TILE, _E), jnp.float32),
            pltpu.VMEM((_TILE, _D), jnp.float32),
            pltpu.VMEM((1, _D), jnp.float32),
            pltpu.VMEM((1, _D), jnp.float32),
        ],
        compiler_params=pltpu.CompilerParams(
            dimension_semantics=("parallel", "arbitrary"),
        ),
    )(xf, gate_w, gate_b2, fc1_w, fc1_b3, ln1_g3, ln1_b3,
      fc2_w, fc2_b3, ln2_g3, ln2_b3)

    output = out.reshape(b, n, d)
    output_top = top.reshape(b, n, d)
    output_bottom = bot.reshape(b, n, d)

    tiles_per_b = num_tiles // b
    sq_b = sq[:, 0, 0].reshape(b, tiles_per_b).sum(axis=1)
    dist = jnp.sqrt(sq_b)
    orthogonality_loss = (1.0 / (dist + 1e-8)).mean()
    return (output, output_top, output_bottom, orthogonality_loss)


# speedup vs baseline: 1.3389x; 1.1635x over previous
import functools

import jax
import jax.numpy as jnp
from jax.experimental import pallas as pl
from jax.experimental.pallas import tpu as pltpu

_E = 8
_K = 2
_D = 768
_TILE = 1024


def _moe_body(x_ref, gate_w_ref, gate_b_ref,
              fc1_w_ref, fc1_b_ref, ln1_g_ref, ln1_b_ref,
              fc2_w_ref, fc2_b_ref, ln2_g_ref, ln2_b_ref,
              out_ref, top_ref, bot_ref, sq_ref,
              acc_top, acc_bot, wt_ref, wb_ref, o_ref, g_ref, b_ref):
    e = pl.program_id(1)
    x = x_ref[...]

    @pl.when(e == 0)
    def _init():
        eye8 = (jax.lax.broadcasted_iota(jnp.int32, (_E, _E), 0) ==
                jax.lax.broadcasted_iota(jnp.int32, (_E, _E), 1)
                ).astype(jnp.float32)
        scores = jax.lax.dot_general(
            x, gate_w_ref[...], (((1,), (1,)), ((), ())),
            preferred_element_type=jnp.float32)
        st = jax.lax.dot_general(
            eye8, scores, (((1,), (1,)), ((), ())),
            precision=jax.lax.Precision.HIGHEST,
            preferred_element_type=jnp.float32) + gate_b_ref[...]
        it = jax.lax.broadcasted_iota(jnp.int32, (_E, _TILE), 0)
        rank_t = jnp.zeros((_E, _TILE), jnp.float32)
        rank_b = jnp.zeros((_E, _TILE), jnp.float32)
        for j in range(_E):
            sj = st[j:j + 1, :]
            eq_earlier = (sj == st) & (it > j)
            rank_t += ((sj > st) | eq_earlier).astype(jnp.float32)
            rank_b += ((sj < st) | eq_earlier).astype(jnp.float32)
        is_top = rank_t < _K
        is_bot = rank_b < _K

        neg_inf = jnp.float32(-1e30)
        m_top = jnp.max(jnp.where(is_top, st, neg_inf), axis=0, keepdims=True)
        e_top = jnp.where(is_top, jnp.exp(st - m_top), 0.0)
        wtT = e_top / jnp.sum(e_top, axis=0, keepdims=True)

        m_bot = jnp.max(jnp.where(is_bot, st, neg_inf), axis=0, keepdims=True)
        e_bot = jnp.where(is_bot, jnp.exp(st - m_bot), 0.0)
        wbT = e_bot / jnp.sum(e_bot, axis=0, keepdims=True)

        wt_ref[...] = jax.lax.dot_general(
            wtT, eye8, (((0,), (0,)), ((), ())),
            precision=jax.lax.Precision.HIGHEST,
            preferred_element_type=jnp.float32)
        wb_ref[...] = jax.lax.dot_general(
            wbT, eye8, (((0,), (0,)), ((), ())),
            precision=jax.lax.Precision.HIGHEST,
            preferred_element_type=jnp.float32)

    def combine(oe, g2, b2, eid, first):
        mu2 = jnp.mean(oe, axis=1, keepdims=True)
        var2 = jnp.mean((oe - mu2) * (oe - mu2), axis=1, keepdims=True)
        y = (oe - mu2) * jax.lax.rsqrt(var2 + 1e-5) * g2 + b2
        onehot = (jax.lax.broadcasted_iota(jnp.int32, (1, _E), 1) == eid
                  ).astype(jnp.float32)
        wt_col = jnp.sum(wt_ref[...] * onehot, axis=1, keepdims=True)
        wb_col = jnp.sum(wb_ref[...] * onehot, axis=1, keepdims=True)
        valid = eid >= 0
        ct = jnp.where(valid, wt_col * y, 0.0)
        cb = jnp.where(valid, wb_col * y, 0.0)
        zero = jnp.float32(0.0)
        acc_top[...] = jnp.where(first, zero, acc_top[...]) + ct
        acc_bot[...] = jnp.where(first, zero, acc_bot[...]) + cb

    combine(o_ref[...], g_ref[...], b_ref[...], e - 1, e == 0)

    w1 = fc1_w_ref[0]
    w2 = fc2_w_ref[0]
    half = _TILE // 2

    def ln1(hh):
        hh = hh + fc1_b_ref[0]
        mu = jnp.mean(hh, axis=1, keepdims=True)
        var = jnp.mean((hh - mu) * (hh - mu), axis=1, keepdims=True)
        hh = (hh - mu) * jax.lax.rsqrt(var + 1e-5) * ln1_g_ref[0] + ln1_b_ref[0]
        return jnp.maximum(hh, 0.0)

    hA = jax.lax.dot_general(x[:half, :], w1, (((1,), (1,)), ((), ())),
                             preferred_element_type=jnp.float32)
    hB = jax.lax.dot_general(x[half:, :], w1, (((1,), (1,)), ((), ())),
                             preferred_element_type=jnp.float32)
    oA = jax.lax.dot_general(ln1(hA), w2, (((1,), (1,)), ((), ())),
                             preferred_element_type=jnp.float32)
    oB = jax.lax.dot_general(ln1(hB), w2, (((1,), (1,)), ((), ())),
                             preferred_element_type=jnp.float32)
    o_ref[:half, :] = oA + fc2_b_ref[0]
    o_ref[half:, :] = oB + fc2_b_ref[0]
    g_ref[...] = ln2_g_ref[0]
    b_ref[...] = ln2_b_ref[0]

    @pl.when(e == _E - 1)
    def _finish():
        combine(o_ref[...], ln2_g_ref[0], ln2_b_ref[0], _E - 1, False)
        top = acc_top[...]
        bot = acc_bot[...]
        top_ref[...] = top
        bot_ref[...] = bot
        out_ref[...] = top + x
        d = top - bot
        sq = jnp.sum(d * d)
        sq_ref[...] = jnp.full((1, 1, 128), sq, dtype=jnp.float32)


@functools.partial(jax.jit, static_argnames=())
def kernel(x, gate_w, gate_b, fc1_w, fc1_b, ln1_g, ln1_b,
           fc2_w, fc2_b, ln2_g, ln2_b):
    b, n, d = x.shape
    bn = b * n
    xf = x.reshape(bn, d)
    num_tiles = bn // _TILE
    gate_b2 = gate_b.reshape(_E, 1)
    fc1_b3 = fc1_b.reshape(_E, 1, d)
    ln1_g3 = ln1_g.reshape(_E, 1, d)
    ln1_b3 = ln1_b.reshape(_E, 1, d)
    fc2_b3 = fc2_b.reshape(_E, 1, d)
    ln2_g3 = ln2_g.reshape(_E, 1, d)
    ln2_b3 = ln2_b.reshape(_E, 1, d)

    grid = (num_tiles, _E)

    def t_only(t, e):
        return (t, 0)

    def e_row3(t, e):
        return (e, 0, 0)

    def const2(t, e):
        return (0, 0)

    out, top, bot, sq = pl.pallas_call(
        _moe_body,
        grid=grid,
        in_specs=[
            pl.BlockSpec((_TILE, d), t_only),
            pl.BlockSpec((_E, d), const2),
            pl.BlockSpec((_E, 1), const2),
            pl.BlockSpec((1, d, d), e_row3),
            pl.BlockSpec((1, 1, d), e_row3),
            pl.BlockSpec((1, 1, d), e_row3),
            pl.BlockSpec((1, 1, d), e_row3),
            pl.BlockSpec((1, d, d), e_row3),
            pl.BlockSpec((1, 1, d), e_row3),
            pl.BlockSpec((1, 1, d), e_row3),
            pl.BlockSpec((1, 1, d), e_row3),
        ],
        out_specs=[
            pl.BlockSpec((_TILE, d), t_only),
            pl.BlockSpec((_TILE, d), t_only),
            pl.BlockSpec((_TILE, d), t_only),
            pl.BlockSpec((1, 1, 128), lambda t, e: (t, 0, 0)),
        ],
        out_shape=[
            jax.ShapeDtypeStruct((bn, d), jnp.float32),
            jax.ShapeDtypeStruct((bn, d), jnp.float32),
            jax.ShapeDtypeStruct((bn, d), jnp.float32),
            jax.ShapeDtypeStruct((num_tiles, 1, 128), jnp.float32),
        ],
        scratch_shapes=[
            pltpu.VMEM((_TILE, _D), jnp.float32),
            pltpu.VMEM((_TILE, _D), jnp.float32),
            pltpu.VMEM((_TILE, _E), jnp.float32),
            pltpu.VMEM((_TILE, _E), jnp.float32),
            pltpu.VMEM((_TILE, _D), jnp.float32),
            pltpu.VMEM((1, _D), jnp.float32),
            pltpu.VMEM((1, _D), jnp.float32),
        ],
        compiler_params=pltpu.CompilerParams(
            dimension_semantics=("parallel", "arbitrary"),
        ),
    )(xf, gate_w, gate_b2, fc1_w, fc1_b3, ln1_g3, ln1_b3,
      fc2_w, fc2_b3, ln2_g3, ln2_b3)

    output = out.reshape(b, n, d)
    output_top = top.reshape(b, n, d)
    output_bottom = bot.reshape(b, n, d)

    tiles_per_b = num_tiles // b
    sq_b = sq[:, 0, 0].reshape(b, tiles_per_b).sum(axis=1)
    dist = jnp.sqrt(sq_b)
    orthogonality_loss = (1.0 / (dist + 1e-8)).mean()
    return (output, output_top, output_bottom, orthogonality_loss)
